# baseline (device time: 19591 ns/iter reference)
import functools

import jax
import jax.numpy as jnp
from jax import lax
from jax.experimental import pallas as pl
from jax.experimental.pallas import tpu as pltpu

N_DEV = 32


def kernel(x):
    m_per, n = x.shape

    def body(x_ref, out_ref, red_ref, send_sems, recv_sems):
        my_pos = lax.axis_index("i")
        peers = [lax.rem(my_pos + d, N_DEV) for d in range(1, N_DEV)]

        barrier_sem = pltpu.get_barrier_semaphore()
        for p in peers:
            pl.semaphore_signal(
                barrier_sem, inc=1,
                device_id=(p,), device_id_type=pl.DeviceIdType.MESH,
            )

        red_ref[pl.ds(my_pos, 1)] = jnp.max(
            x_ref[...], axis=0, keepdims=True
        )[None]

        pl.semaphore_wait(barrier_sem, N_DEV - 1)

        rdmas = []
        for p in peers:
            rdma = pltpu.make_async_remote_copy(
                src_ref=red_ref.at[pl.ds(my_pos, 1)],
                dst_ref=red_ref.at[pl.ds(my_pos, 1)],
                send_sem=send_sems.at[p],
                recv_sem=recv_sems.at[my_pos],
                device_id=(p,),
                device_id_type=pl.DeviceIdType.MESH,
            )
            rdma.start()
            rdmas.append(rdma)

        for p in peers:
            recv = pltpu.make_async_remote_copy(
                src_ref=red_ref.at[pl.ds(p, 1)],
                dst_ref=red_ref.at[pl.ds(p, 1)],
                send_sem=send_sems.at[p],
                recv_sem=recv_sems.at[p],
                device_id=(my_pos,),
                device_id_type=pl.DeviceIdType.MESH,
            )
            recv.wait_recv()

        for rdma in rdmas:
            rdma.wait_send()

        @functools.partial(pl.run_scoped, sem=pltpu.SemaphoreType.REGULAR)
        def _(sem):
            for p in peers:
                pl.semaphore_signal(
                    sem, inc=1,
                    device_id=(p,), device_id_type=pl.DeviceIdType.MESH,
                )
            out_ref[...] = jnp.max(red_ref[...], axis=0)
            pl.semaphore_wait(sem, N_DEV - 1)

    return pl.pallas_call(
        body,
        out_shape=jax.ShapeDtypeStruct((1, n), x.dtype),
        in_specs=[pl.BlockSpec(memory_space=pltpu.VMEM)],
        out_specs=pl.BlockSpec(memory_space=pltpu.VMEM),
        scratch_shapes=[
            pltpu.VMEM((N_DEV, 1, n), x.dtype),
            pltpu.SemaphoreType.DMA((N_DEV,)),
            pltpu.SemaphoreType.DMA((N_DEV,)),
        ],
        compiler_params=pltpu.CompilerParams(collective_id=0),
    )(x)


# device time: 13264 ns/iter; 1.4770x vs baseline; 1.4770x over previous
import jax
import jax.numpy as jnp
from jax import lax
from jax.experimental import pallas as pl
from jax.experimental.pallas import tpu as pltpu

N_DEV = 32


def kernel(x):
    m_per, n = x.shape

    def body(x_ref, out_ref, red_ref, send_sems, recv_sems):
        my_pos = lax.axis_index("i")
        peers = [lax.rem(my_pos + d, N_DEV) for d in range(1, N_DEV)]

        barrier_sem = pltpu.get_barrier_semaphore()
        for p in peers:
            pl.semaphore_signal(
                barrier_sem, inc=1,
                device_id=(p,), device_id_type=pl.DeviceIdType.MESH,
            )

        red_ref[pl.ds(my_pos, 1)] = jnp.max(
            x_ref[...], axis=0, keepdims=True
        )[None]

        pl.semaphore_wait(barrier_sem, N_DEV - 1)

        rdmas = []
        for p in peers:
            rdma = pltpu.make_async_remote_copy(
                src_ref=red_ref.at[pl.ds(my_pos, 1)],
                dst_ref=red_ref.at[pl.ds(my_pos, 1)],
                send_sem=send_sems.at[p],
                recv_sem=recv_sems.at[my_pos],
                device_id=(p,),
                device_id_type=pl.DeviceIdType.MESH,
            )
            rdma.start()
            rdmas.append(rdma)

        for p in peers:
            recv = pltpu.make_async_remote_copy(
                src_ref=red_ref.at[pl.ds(p, 1)],
                dst_ref=red_ref.at[pl.ds(p, 1)],
                send_sem=send_sems.at[p],
                recv_sem=recv_sems.at[p],
                device_id=(my_pos,),
                device_id_type=pl.DeviceIdType.MESH,
            )
            recv.wait_recv()

        for rdma in rdmas:
            rdma.wait_send()

        out_ref[...] = jnp.max(red_ref[...], axis=0)

    return pl.pallas_call(
        body,
        out_shape=jax.ShapeDtypeStruct((1, n), x.dtype),
        in_specs=[pl.BlockSpec(memory_space=pltpu.VMEM)],
        out_specs=pl.BlockSpec(memory_space=pltpu.VMEM),
        scratch_shapes=[
            pltpu.VMEM((N_DEV, 1, n), x.dtype),
            pltpu.SemaphoreType.DMA((N_DEV,)),
            pltpu.SemaphoreType.DMA((N_DEV,)),
        ],
        compiler_params=pltpu.CompilerParams(collective_id=0),
    )(x)
